# baseline (speedup 1.0000x reference)
import dataclasses
import functools

import jax
import jax.numpy as jnp
from jax import lax
from jax.experimental import pallas as pl
from jax.experimental.pallas import tpu as pltpu
from jax.experimental.pallas import tpu_sc as plsc

N = 10000
NP = 10240
E = 160000
E2 = 176000
NT = 16
NC = 2
EP_T = E2 // NT
KCH = 125
NCH = EP_T // KCH
RSTR = NP // NT
WCH = 128
SW = 64
RB = 400
GRID = N // RB


def _sc_params():
    cp = pltpu.CompilerParams(use_tc_tiling_on_sc=False)
    if "needs_layout_passes" in pltpu.CompilerParams.__dataclass_fields__:
        cp = dataclasses.replace(cp, needs_layout_passes=False)
    return cp


_MESH = plsc.VectorSubcoreMesh(core_axis_name="c", subcore_axis_name="s")


def _deg_call(dst_r):
    @functools.partial(
        pl.kernel,
        out_type=jax.ShapeDtypeStruct((NT, RSTR, 16), jnp.float32),
        mesh=_MESH,
        compiler_params=_sc_params(),
        scratch_types=[
            pltpu.VMEM((NCH, KCH), jnp.int32),
            pltpu.VMEM((KCH, 16), jnp.float32),
            pltpu.VMEM((RSTR, 16), jnp.float32),
            pltpu.VMEM_SHARED((NP, 16), jnp.float32),
        ],
    )
    def k(dst_hbm, deg_hbm, idx_v, ones_v, stripe_v, acc_sh):
        c = lax.axis_index("c")
        s = lax.axis_index("s")

        @pl.loop(0, RSTR)
        def _(r):
            stripe_v[r] = jnp.zeros((16,), jnp.float32)

        @pl.loop(0, KCH)
        def _(r):
            ones_v[r] = jnp.ones((16,), jnp.float32)

        pltpu.sync_copy(stripe_v, acc_sh.at[pl.ds(s * RSTR, RSTR)])
        pltpu.sync_copy(dst_hbm.at[s], idx_v)
        plsc.subcore_barrier()

        @pl.loop(0, NCH)
        def _(j):
            pltpu.sync_copy(ones_v, acc_sh.at[idx_v.at[j]], add=True)

        plsc.subcore_barrier()

        @pl.when(c == 0)
        def _():
            pltpu.sync_copy(acc_sh.at[pl.ds(s * RSTR, RSTR)], stripe_v)
            pltpu.sync_copy(stripe_v, deg_hbm.at[s])

    return k(dst_r)


def _agg_call(y_slabs, src_r, dst_r, n_slabs):
    nspc = n_slabs // NC

    @functools.partial(
        pl.kernel,
        out_type=jax.ShapeDtypeStruct((n_slabs, NP, SW), jnp.float32),
        mesh=_MESH,
        compiler_params=_sc_params(),
        scratch_types=[
            pltpu.VMEM((NCH, KCH), jnp.int32),
            pltpu.VMEM((NCH, KCH), jnp.int32),
            pltpu.VMEM((KCH, SW), jnp.float32),
            pltpu.VMEM((KCH, SW), jnp.float32),
            pltpu.VMEM((KCH, SW), jnp.float32),
            pltpu.VMEM((KCH, SW), jnp.float32),
            pltpu.VMEM((WCH, SW), jnp.float32),
            pltpu.VMEM((WCH, SW), jnp.float32),
            pltpu.SemaphoreType.DMA,
            pltpu.SemaphoreType.DMA,
            pltpu.SemaphoreType.DMA,
            pltpu.SemaphoreType.DMA,
            pltpu.SemaphoreType.DMA,
            pltpu.SemaphoreType.DMA,
            pltpu.SemaphoreType.DMA,
            pltpu.SemaphoreType.DMA,
            pltpu.VMEM_SHARED((NP, SW), jnp.float32),
        ],
    )
    def k(y_hbm, src_hbm, dst_hbm, out_hbm, src_v, dst_v, rows0_v, rows1_v,
          rows2_v, rows3_v, zb_v, wb_v, gsem0, gsem1, gsem2, gsem3,
          ssem0, ssem1, ssem2, ssem3, acc_sh):
        c = lax.axis_index("c")
        s = lax.axis_index("s")

        pltpu.sync_copy(src_hbm.at[s], src_v)
        pltpu.sync_copy(dst_hbm.at[s], dst_v)

        @pl.loop(0, WCH)
        def _(r):
            for kk in range(SW // 16):
                zb_v[r, pl.ds(16 * kk, 16)] = jnp.zeros((16,), jnp.float32)

        for jj in range(nspc):
            slab = c * nspc + jj

            def g_start(j, buf, sem):
                pltpu.async_copy(y_hbm.at[slab].at[src_v.at[j]], buf, sem)

            def g_wait(buf, sem):
                pltpu.make_async_copy(
                    y_hbm.at[slab].at[src_v.at[0]], buf, sem).wait()

            def s_start(j, buf, sem):
                pltpu.async_copy(buf, acc_sh.at[dst_v.at[j]], sem, add=True)

            def s_wait(buf, sem):
                pltpu.make_async_copy(
                    buf, acc_sh.at[dst_v.at[0]], sem).wait()

            for p in range(RSTR // WCH):
                pltpu.sync_copy(zb_v, acc_sh.at[pl.ds(s * RSTR + p * WCH, WCH)])
            plsc.subcore_barrier()

            bufs = (rows0_v, rows1_v, rows2_v, rows3_v)
            gsems = (gsem0, gsem1, gsem2, gsem3)
            ssems = (ssem0, ssem1, ssem2, ssem3)

            g_start(0, bufs[0], gsems[0])
            g_start(1, bufs[1], gsems[1])
            g_start(2, bufs[2], gsems[2])
            for cc in range(4):
                bk = cc % 4
                g_wait(bufs[bk], gsems[bk])
                s_start(cc, bufs[bk], ssems[bk])
                nxt = (cc + 3) % 4
                if cc == 0:
                    g_start(3, bufs[3], gsems[3])
                else:
                    s_wait(bufs[nxt], ssems[nxt])
                    g_start(cc + 3, bufs[nxt], gsems[nxt])

            @pl.loop(1, NCH // 4 - 1)
            def _(h):
                for kk in range(4):
                    cc = 4 * h + kk
                    g_wait(bufs[kk], gsems[kk])
                    s_start(cc, bufs[kk], ssems[kk])
                    nxt = (kk + 3) % 4
                    s_wait(bufs[nxt], ssems[nxt])
                    g_start(cc + 3, bufs[nxt], gsems[nxt])

            for kk in range(4):
                cc = NCH - 4 + kk
                g_wait(bufs[kk], gsems[kk])
                s_start(cc, bufs[kk], ssems[kk])
                if kk == 0:
                    nxt = 3
                    s_wait(bufs[nxt], ssems[nxt])
                    g_start(cc + 3, bufs[nxt], gsems[nxt])
            for kk in range(4):
                s_wait(bufs[kk], ssems[kk])
            plsc.subcore_barrier()
            for p in range(RSTR // WCH):
                r0 = s * RSTR + p * WCH
                pltpu.sync_copy(acc_sh.at[pl.ds(r0, WCH)], wb_v)
                pltpu.sync_copy(wb_v, out_hbm.at[slab].at[pl.ds(r0, WCH)])
            plsc.subcore_barrier()

    return k(y_slabs, src_r, dst_r)


def _edge_mask_call(node_mask_flat, src_flat, dst_flat):
    ep_w = E // (NT * NC)
    nchunk = ep_w // 16 + 1

    @functools.partial(
        pl.kernel,
        out_type=jax.ShapeDtypeStruct((E,), jnp.float32),
        mesh=_MESH,
        compiler_params=_sc_params(),
        scratch_types=[
            pltpu.VMEM((N,), jnp.float32),
            pltpu.VMEM((nchunk * 16,), jnp.int32),
            pltpu.VMEM((nchunk * 16,), jnp.int32),
            pltpu.VMEM((nchunk * 16,), jnp.float32),
        ],
    )
    def k(nm_hbm, src_hbm, dst_hbm, out_hbm, tab_v, src_v, dst_v, out_v):
        c = lax.axis_index("c")
        s = lax.axis_index("s")
        wid = s * NC + c
        base = wid * ep_w

        pltpu.sync_copy(nm_hbm, tab_v)
        src_v[pl.ds(ep_w - 8, 16)] = jnp.zeros((16,), jnp.int32)
        dst_v[pl.ds(ep_w - 8, 16)] = jnp.zeros((16,), jnp.int32)
        pltpu.sync_copy(src_hbm.at[pl.ds(base, ep_w)], src_v.at[pl.ds(0, ep_w)])
        pltpu.sync_copy(dst_hbm.at[pl.ds(base, ep_w)], dst_v.at[pl.ds(0, ep_w)])

        @pl.loop(0, nchunk)
        def _(i):
            sl = pl.ds(i * 16, 16)
            a = plsc.load_gather(tab_v, [src_v[sl]])
            b = plsc.load_gather(tab_v, [dst_v[sl]])
            out_v[sl] = 0.5 * (a + b)

        pltpu.sync_copy(out_v.at[pl.ds(0, ep_w)], out_hbm.at[pl.ds(base, ep_w)])

    return k(node_mask_flat, src_flat, dst_flat)


def _mm1_body(x_ref, w_ref, deg_ref, y1_ref, dis_ref):
    xw = jnp.dot(x_ref[...].astype(jnp.bfloat16), w_ref[...].astype(jnp.bfloat16),
                 preferred_element_type=jnp.float32)
    dis = lax.rsqrt(deg_ref[:, 0:1])
    y = xw * dis
    for kk in range(8):
        y1_ref[kk] = y[:, SW * kk:SW * (kk + 1)]
    dis_ref[...] = dis


def _mm1_call(x, W1, deg16):
    return pl.pallas_call(
        _mm1_body,
        grid=(GRID,),
        in_specs=[
            pl.BlockSpec((RB, 1280), lambda i: (i, 0)),
            pl.BlockSpec((1280, 512), lambda i: (0, 0)),
            pl.BlockSpec((RB, 16), lambda i: (i, 0)),
        ],
        out_specs=[
            pl.BlockSpec((8, RB, SW), lambda i: (0, i, 0)),
            pl.BlockSpec((RB, 1), lambda i: (i, 0)),
        ],
        out_shape=[
            jax.ShapeDtypeStruct((8, N, SW), jnp.float32),
            jax.ShapeDtypeStruct((N, 1), jnp.float32),
        ],
    )(x, W1, deg16)


def _mm2_body(agg_ref, dis_ref, w2_ref, b1_ref, y2_ref):
    agg = jnp.concatenate([agg_ref[kk] for kk in range(8)], axis=-1)
    dis = dis_ref[...]
    h1 = jnp.maximum(agg * dis + b1_ref[...], 0.0)
    xw2 = jnp.dot(h1.astype(jnp.bfloat16), w2_ref[...].astype(jnp.bfloat16),
                  preferred_element_type=jnp.float32)
    y2 = xw2 * dis
    for kk in range(4):
        y2_ref[kk] = y2[:, SW * kk:SW * (kk + 1)]


def _mm2_call(agg1, dis, W2, b1r):
    return pl.pallas_call(
        _mm2_body,
        grid=(GRID,),
        in_specs=[
            pl.BlockSpec((8, RB, SW), lambda i: (0, i, 0)),
            pl.BlockSpec((RB, 1), lambda i: (i, 0)),
            pl.BlockSpec((512, 256), lambda i: (0, 0)),
            pl.BlockSpec((1, 512), lambda i: (0, 0)),
        ],
        out_specs=[pl.BlockSpec((4, RB, SW), lambda i: (0, i, 0))],
        out_shape=[jax.ShapeDtypeStruct((4, N, SW), jnp.float32)],
    )(agg1, dis, W2, b1r)[0]


def _head_body(agg_ref, dis_ref, b2_ref, wl_ref, bl_ref, eps_ref,
               nm_ref, kld_ref):
    agg = jnp.concatenate([agg_ref[kk] for kk in range(4)], axis=-1)
    dis = dis_ref[...]
    h2 = jnp.maximum(agg * dis + b2_ref[...], 0.0)
    pre = jnp.dot(h2, wl_ref[...], preferred_element_type=jnp.float32)
    pre = pre + bl_ref[...]
    pre = jnp.clip(pre, -10.0, 10.0)
    eps = eps_ref[...]
    gate = jnp.log(eps) - jnp.log(1.0 - eps) + pre
    nm = jax.nn.sigmoid(gate)
    nm_ref[...] = nm
    ee = 1e-08
    t = nm * jnp.log(nm / 0.5 + ee) + (1.0 - nm) * jnp.log((1.0 - nm) / 0.5 + ee)
    partial = jnp.full((1, 1), jnp.sum(t) * (1.0 / N), jnp.float32)

    @pl.when(pl.program_id(0) == 0)
    def _():
        kld_ref[...] = jnp.zeros((1, 1), jnp.float32)

    kld_ref[...] += partial


def _head_call(agg2, dis, b2r, Wl, blr, eps):
    return pl.pallas_call(
        _head_body,
        grid=(GRID,),
        in_specs=[
            pl.BlockSpec((4, RB, SW), lambda i: (0, i, 0)),
            pl.BlockSpec((RB, 1), lambda i: (i, 0)),
            pl.BlockSpec((1, 256), lambda i: (0, 0)),
            pl.BlockSpec((256, 1), lambda i: (0, 0)),
            pl.BlockSpec((1, 1), lambda i: (0, 0)),
            pl.BlockSpec((RB, 1), lambda i: (i, 0)),
        ],
        out_specs=[
            pl.BlockSpec((RB, 1), lambda i: (i, 0)),
            pl.BlockSpec((1, 1), lambda i: (0, 0)),
        ],
        out_shape=[
            jax.ShapeDtypeStruct((N, 1), jnp.float32),
            jax.ShapeDtypeStruct((1, 1), jnp.float32),
        ],
    )(agg2, dis, b2r, Wl, blr, eps)


@jax.jit
def kernel(x, edge_index, W1, b1, W2, b2, Wl, bl, eps):
    src = edge_index[0]
    dst = edge_index[1]
    loops = jnp.arange(N, dtype=jnp.int32)
    padi = jnp.zeros((E2 - E - N,), jnp.int32)
    src_r = jnp.concatenate([src, loops, padi]).reshape(NT, NCH, KCH)
    dst_r = jnp.concatenate([dst, loops, padi + (N + 64)]).reshape(NT, NCH, KCH)

    deg16 = _deg_call(dst_r).reshape(NP, 16)
    y1, dis = _mm1_call(x, W1, deg16)
    agg1 = _agg_call(y1, src_r, dst_r, 8)
    y2 = _mm2_call(agg1, dis, W2, b1.reshape(1, 512))
    agg2 = _agg_call(y2, src_r, dst_r, 4)
    node_mask, kld = _head_call(agg2, dis, b2.reshape(1, 256), Wl,
                                bl.reshape(1, 1), eps)
    edge_mask = _edge_mask_call(node_mask.reshape(N), src, dst)
    return kld[0, 0], node_mask, edge_mask[:, None]


# speedup vs baseline: 1.0021x; 1.0021x over previous
import dataclasses
import functools

import jax
import jax.numpy as jnp
from jax import lax
from jax.experimental import pallas as pl
from jax.experimental.pallas import tpu as pltpu
from jax.experimental.pallas import tpu_sc as plsc

N = 10000
NP = 10240
E = 160000
E2 = 176000
NT = 16
NC = 2
EP_T = E2 // NT
KCH = 125
NCH = EP_T // KCH
RSTR = NP // NT
WCH = 128
SW = 64
RB = 400
GRID = N // RB


def _sc_params():
    cp = pltpu.CompilerParams(use_tc_tiling_on_sc=False)
    if "needs_layout_passes" in pltpu.CompilerParams.__dataclass_fields__:
        cp = dataclasses.replace(cp, needs_layout_passes=False)
    return cp


_MESH = plsc.VectorSubcoreMesh(core_axis_name="c", subcore_axis_name="s")


def _deg_call(dst_r):
    @functools.partial(
        pl.kernel,
        out_type=jax.ShapeDtypeStruct((NT, RSTR, 16), jnp.float32),
        mesh=_MESH,
        compiler_params=_sc_params(),
        scratch_types=[
            pltpu.VMEM((NCH, KCH), jnp.int32),
            pltpu.VMEM((KCH, 16), jnp.float32),
            pltpu.VMEM((RSTR, 16), jnp.float32),
            pltpu.VMEM_SHARED((NP, 16), jnp.float32),
        ],
    )
    def k(dst_hbm, deg_hbm, idx_v, ones_v, stripe_v, acc_sh):
        c = lax.axis_index("c")
        s = lax.axis_index("s")

        @pl.loop(0, RSTR)
        def _(r):
            stripe_v[r] = jnp.zeros((16,), jnp.float32)

        @pl.loop(0, KCH)
        def _(r):
            ones_v[r] = jnp.ones((16,), jnp.float32)

        pltpu.sync_copy(stripe_v, acc_sh.at[pl.ds(s * RSTR, RSTR)])
        pltpu.sync_copy(dst_hbm.at[s], idx_v)
        plsc.subcore_barrier()

        @pl.loop(0, NCH)
        def _(j):
            pltpu.sync_copy(ones_v, acc_sh.at[idx_v.at[j]], add=True)

        plsc.subcore_barrier()

        @pl.when(c == 0)
        def _():
            pltpu.sync_copy(acc_sh.at[pl.ds(s * RSTR, RSTR)], stripe_v)
            pltpu.sync_copy(stripe_v, deg_hbm.at[s])

    return k(dst_r)


def _agg_call(y_slabs, src_r, dst_r, n_slabs):
    nspc = n_slabs // NC

    @functools.partial(
        pl.kernel,
        out_type=jax.ShapeDtypeStruct((n_slabs, NP, SW), jnp.float32),
        mesh=_MESH,
        compiler_params=_sc_params(),
        scratch_types=[
            pltpu.VMEM((NCH, KCH), jnp.int32),
            pltpu.VMEM((NCH, KCH), jnp.int32),
            pltpu.VMEM((KCH, SW), jnp.float32),
            pltpu.VMEM((KCH, SW), jnp.float32),
            pltpu.VMEM((KCH, SW), jnp.float32),
            pltpu.VMEM((KCH, SW), jnp.float32),
            pltpu.VMEM((WCH, SW), jnp.float32),
            pltpu.VMEM((WCH, SW), jnp.float32),
            pltpu.SemaphoreType.DMA,
            pltpu.SemaphoreType.DMA,
            pltpu.SemaphoreType.DMA,
            pltpu.SemaphoreType.DMA,
            pltpu.SemaphoreType.DMA,
            pltpu.SemaphoreType.DMA,
            pltpu.SemaphoreType.DMA,
            pltpu.SemaphoreType.DMA,
            pltpu.VMEM_SHARED((NP, SW), jnp.float32),
        ],
    )
    def k(y_hbm, src_hbm, dst_hbm, out_hbm, src_v, dst_v, rows0_v, rows1_v,
          rows2_v, rows3_v, zb_v, wb_v, gsem0, gsem1, gsem2, gsem3,
          ssem0, ssem1, ssem2, ssem3, acc_sh):
        c = lax.axis_index("c")
        s = lax.axis_index("s")

        pltpu.sync_copy(src_hbm.at[s], src_v)
        pltpu.sync_copy(dst_hbm.at[s], dst_v)

        @pl.loop(0, WCH)
        def _(r):
            for kk in range(SW // 16):
                zb_v[r, pl.ds(16 * kk, 16)] = jnp.zeros((16,), jnp.float32)

        for jj in range(nspc):
            slab = c * nspc + jj

            def g_start(j, buf, sem):
                pltpu.async_copy(y_hbm.at[slab].at[src_v.at[j]], buf, sem)

            def g_wait(buf, sem):
                pltpu.make_async_copy(
                    y_hbm.at[slab].at[src_v.at[0]], buf, sem).wait()

            def s_start(j, buf, sem):
                pltpu.async_copy(buf, acc_sh.at[dst_v.at[j]], sem, add=True)

            def s_wait(buf, sem):
                pltpu.make_async_copy(
                    buf, acc_sh.at[dst_v.at[0]], sem).wait()

            for p in range(RSTR // WCH):
                pltpu.sync_copy(zb_v, acc_sh.at[pl.ds(s * RSTR + p * WCH, WCH)])
            plsc.subcore_barrier()

            bufs = (rows0_v, rows1_v, rows2_v, rows3_v)
            gsems = (gsem0, gsem1, gsem2, gsem3)
            ssems = (ssem0, ssem1, ssem2, ssem3)

            g_start(0, bufs[0], gsems[0])
            g_start(1, bufs[1], gsems[1])
            g_start(2, bufs[2], gsems[2])
            for cc in range(4):
                bk = cc % 4
                g_wait(bufs[bk], gsems[bk])
                s_start(cc, bufs[bk], ssems[bk])
                nxt = (cc + 3) % 4
                if cc == 0:
                    g_start(3, bufs[3], gsems[3])
                else:
                    s_wait(bufs[nxt], ssems[nxt])
                    g_start(cc + 3, bufs[nxt], gsems[nxt])

            @pl.loop(1, NCH // 4 - 1)
            def _(h):
                for kk in range(4):
                    cc = 4 * h + kk
                    g_wait(bufs[kk], gsems[kk])
                    s_start(cc, bufs[kk], ssems[kk])
                    nxt = (kk + 3) % 4
                    s_wait(bufs[nxt], ssems[nxt])
                    g_start(cc + 3, bufs[nxt], gsems[nxt])

            for kk in range(4):
                cc = NCH - 4 + kk
                g_wait(bufs[kk], gsems[kk])
                s_start(cc, bufs[kk], ssems[kk])
                if kk == 0:
                    nxt = 3
                    s_wait(bufs[nxt], ssems[nxt])
                    g_start(cc + 3, bufs[nxt], gsems[nxt])
            for kk in range(4):
                s_wait(bufs[kk], ssems[kk])
            plsc.subcore_barrier()
            for p in range(RSTR // WCH):
                r0 = s * RSTR + p * WCH
                pltpu.sync_copy(acc_sh.at[pl.ds(r0, WCH)], wb_v)
                pltpu.sync_copy(wb_v, out_hbm.at[slab].at[pl.ds(r0, WCH)])
            plsc.subcore_barrier()

    return k(y_slabs, src_r, dst_r)


def _edge_mask_call(node_mask_flat, src_flat, dst_flat):
    ep_w = E // (NT * NC)
    nchunk = ep_w // 16 + 1

    @functools.partial(
        pl.kernel,
        out_type=jax.ShapeDtypeStruct((E,), jnp.float32),
        mesh=_MESH,
        compiler_params=_sc_params(),
        scratch_types=[
            pltpu.VMEM((N,), jnp.float32),
            pltpu.VMEM((nchunk * 16,), jnp.int32),
            pltpu.VMEM((nchunk * 16,), jnp.int32),
            pltpu.VMEM((nchunk * 16,), jnp.float32),
        ],
    )
    def k(nm_hbm, src_hbm, dst_hbm, out_hbm, tab_v, src_v, dst_v, out_v):
        c = lax.axis_index("c")
        s = lax.axis_index("s")
        wid = s * NC + c
        base = wid * ep_w

        pltpu.sync_copy(nm_hbm, tab_v)
        src_v[pl.ds(ep_w - 8, 16)] = jnp.zeros((16,), jnp.int32)
        dst_v[pl.ds(ep_w - 8, 16)] = jnp.zeros((16,), jnp.int32)
        pltpu.sync_copy(src_hbm.at[pl.ds(base, ep_w)], src_v.at[pl.ds(0, ep_w)])
        pltpu.sync_copy(dst_hbm.at[pl.ds(base, ep_w)], dst_v.at[pl.ds(0, ep_w)])

        @pl.loop(0, nchunk)
        def _(i):
            sl = pl.ds(i * 16, 16)
            a = plsc.load_gather(tab_v, [src_v[sl]])
            b = plsc.load_gather(tab_v, [dst_v[sl]])
            out_v[sl] = 0.5 * (a + b)

        pltpu.sync_copy(out_v.at[pl.ds(0, ep_w)], out_hbm.at[pl.ds(base, ep_w)])

    return k(node_mask_flat, src_flat, dst_flat)


def _mm1_body(x_ref, w_ref, deg_ref, y1_ref, dis_ref):
    xw = jnp.dot(x_ref[...].astype(jnp.bfloat16), w_ref[...].astype(jnp.bfloat16),
                 preferred_element_type=jnp.float32)
    dis = lax.rsqrt(deg_ref[:, 0:1])
    y = xw * dis
    for kk in range(8):
        y1_ref[kk] = y[:, SW * kk:SW * (kk + 1)]
    dis_ref[...] = dis


def _mm1_call(x, W1, deg16):
    return pl.pallas_call(
        _mm1_body,
        grid=(GRID,),
        in_specs=[
            pl.BlockSpec((RB, 1280), lambda i: (i, 0)),
            pl.BlockSpec((1280, 512), lambda i: (0, 0)),
            pl.BlockSpec((RB, 16), lambda i: (i, 0)),
        ],
        out_specs=[
            pl.BlockSpec((8, RB, SW), lambda i: (0, i, 0)),
            pl.BlockSpec((RB, 1), lambda i: (i, 0)),
        ],
        out_shape=[
            jax.ShapeDtypeStruct((8, N, SW), jnp.float32),
            jax.ShapeDtypeStruct((N, 1), jnp.float32),
        ],
    )(x, W1, deg16)


def _mm2_body(agg_ref, dis_ref, w2_ref, b1_ref, y2_ref):
    agg = jnp.concatenate([agg_ref[kk] for kk in range(8)], axis=-1)
    dis = dis_ref[...]
    h1 = jnp.maximum(agg * dis + b1_ref[...], 0.0)
    xw2 = jnp.dot(h1.astype(jnp.bfloat16), w2_ref[...].astype(jnp.bfloat16),
                  preferred_element_type=jnp.float32)
    y2 = xw2 * dis
    for kk in range(4):
        y2_ref[kk] = y2[:, SW * kk:SW * (kk + 1)]


def _mm2_call(agg1, dis, W2, b1r):
    return pl.pallas_call(
        _mm2_body,
        grid=(GRID,),
        in_specs=[
            pl.BlockSpec((8, RB, SW), lambda i: (0, i, 0)),
            pl.BlockSpec((RB, 1), lambda i: (i, 0)),
            pl.BlockSpec((512, 256), lambda i: (0, 0)),
            pl.BlockSpec((1, 512), lambda i: (0, 0)),
        ],
        out_specs=[pl.BlockSpec((4, RB, SW), lambda i: (0, i, 0))],
        out_shape=[jax.ShapeDtypeStruct((4, N, SW), jnp.float32)],
    )(agg1, dis, W2, b1r)[0]


def _head_body(agg_ref, dis_ref, b2_ref, wl_ref, bl_ref, eps_ref,
               nm_ref, kld_ref):
    agg = jnp.concatenate([agg_ref[kk] for kk in range(4)], axis=-1)
    dis = dis_ref[...]
    h2 = jnp.maximum(agg * dis + b2_ref[...], 0.0)
    pre = jnp.dot(h2, wl_ref[...], preferred_element_type=jnp.float32)
    pre = pre + bl_ref[...]
    pre = jnp.clip(pre, -10.0, 10.0)
    eps = eps_ref[...]
    gate = jnp.log(eps) - jnp.log(1.0 - eps) + pre
    nm = jax.nn.sigmoid(gate)
    nm_ref[...] = nm
    ee = 1e-08
    t = nm * jnp.log(nm / 0.5 + ee) + (1.0 - nm) * jnp.log((1.0 - nm) / 0.5 + ee)
    partial = jnp.full((1, 1), jnp.sum(t) * (1.0 / N), jnp.float32)

    @pl.when(pl.program_id(0) == 0)
    def _():
        kld_ref[...] = jnp.zeros((1, 1), jnp.float32)

    kld_ref[...] += partial


def _head_call(agg2, dis, b2r, Wl, blr, eps):
    return pl.pallas_call(
        _head_body,
        grid=(GRID,),
        in_specs=[
            pl.BlockSpec((4, RB, SW), lambda i: (0, i, 0)),
            pl.BlockSpec((RB, 1), lambda i: (i, 0)),
            pl.BlockSpec((1, 256), lambda i: (0, 0)),
            pl.BlockSpec((256, 1), lambda i: (0, 0)),
            pl.BlockSpec((1, 1), lambda i: (0, 0)),
            pl.BlockSpec((RB, 1), lambda i: (i, 0)),
        ],
        out_specs=[
            pl.BlockSpec((RB, 1), lambda i: (i, 0)),
            pl.BlockSpec((1, 1), lambda i: (0, 0)),
        ],
        out_shape=[
            jax.ShapeDtypeStruct((N, 1), jnp.float32),
            jax.ShapeDtypeStruct((1, 1), jnp.float32),
        ],
    )(agg2, dis, b2r, Wl, blr, eps)


@jax.jit
def kernel(x, edge_index, W1, b1, W2, b2, Wl, bl, eps):
    src = edge_index[0]
    dst = edge_index[1]
    loops = jnp.arange(N, dtype=jnp.int32)
    padi = jnp.zeros((E2 - E - N,), jnp.int32)
    padd = N + (jnp.arange(E2 - E - N, dtype=jnp.int32) % (NP - N))
    src_r = jnp.concatenate([src, loops, padi]).reshape(NT, NCH, KCH)
    dst_r = jnp.concatenate([dst, loops, padd]).reshape(NT, NCH, KCH)

    deg16 = _deg_call(dst_r).reshape(NP, 16)
    y1, dis = _mm1_call(x, W1, deg16)
    agg1 = _agg_call(y1, src_r, dst_r, 8)
    y2 = _mm2_call(agg1, dis, W2, b1.reshape(1, 512))
    agg2 = _agg_call(y2, src_r, dst_r, 4)
    node_mask, kld = _head_call(agg2, dis, b2.reshape(1, 256), Wl,
                                bl.reshape(1, 1), eps)
    edge_mask = _edge_mask_call(node_mask.reshape(N), src, dst)
    return kld[0, 0], node_mask, edge_mask[:, None]


# speedup vs baseline: 2.4448x; 2.4398x over previous
import dataclasses
import functools

import jax
import jax.numpy as jnp
from jax import lax
from jax.experimental import pallas as pl
from jax.experimental.pallas import tpu as pltpu
from jax.experimental.pallas import tpu_sc as plsc

N = 10000
NP = 10240
E = 160000
E2 = 176000
NT = 16
NC = 2
EP_T = E2 // NT
KCH = 125
NCH = EP_T // KCH
RSTR = NP // NT
WCH = 128
SW = 64
RB = 400
GRID = N // RB


def _sc_params():
    cp = pltpu.CompilerParams(use_tc_tiling_on_sc=False)
    if "needs_layout_passes" in pltpu.CompilerParams.__dataclass_fields__:
        cp = dataclasses.replace(cp, needs_layout_passes=False)
    return cp


_MESH = plsc.VectorSubcoreMesh(core_axis_name="c", subcore_axis_name="s")


def _deg_call(dst_r):
    @functools.partial(
        pl.kernel,
        out_type=jax.ShapeDtypeStruct((NT, RSTR, 16), jnp.float32),
        mesh=_MESH,
        compiler_params=_sc_params(),
        scratch_types=[
            pltpu.VMEM((NCH, KCH), jnp.int32),
            pltpu.VMEM((KCH, 16), jnp.float32),
            pltpu.VMEM((RSTR, 16), jnp.float32),
            pltpu.VMEM_SHARED((NP, 16), jnp.float32),
        ],
    )
    def k(dst_hbm, deg_hbm, idx_v, ones_v, stripe_v, acc_sh):
        c = lax.axis_index("c")
        s = lax.axis_index("s")

        @pl.loop(0, RSTR)
        def _(r):
            stripe_v[r] = jnp.zeros((16,), jnp.float32)

        @pl.loop(0, KCH)
        def _(r):
            ones_v[r] = jnp.ones((16,), jnp.float32)

        pltpu.sync_copy(stripe_v, acc_sh.at[pl.ds(s * RSTR, RSTR)])
        pltpu.sync_copy(dst_hbm.at[s], idx_v)
        plsc.subcore_barrier()

        @pl.loop(0, NCH)
        def _(j):
            pltpu.sync_copy(ones_v, acc_sh.at[idx_v.at[j]], add=True)

        plsc.subcore_barrier()

        @pl.when(c == 0)
        def _():
            pltpu.sync_copy(acc_sh.at[pl.ds(s * RSTR, RSTR)], stripe_v)
            pltpu.sync_copy(stripe_v, deg_hbm.at[s])

    return k(dst_r)


def _agg_call(y_slabs, src_r, dst_r, n_slabs):
    nspc = n_slabs // NC

    @functools.partial(
        pl.kernel,
        out_type=jax.ShapeDtypeStruct((n_slabs, NP, SW), jnp.float32),
        mesh=_MESH,
        compiler_params=_sc_params(),
        scratch_types=[
            pltpu.VMEM((NCH, KCH), jnp.int32),
            pltpu.VMEM((NCH, KCH), jnp.int32),
            pltpu.VMEM((KCH, SW), jnp.float32),
            pltpu.VMEM((KCH, SW), jnp.float32),
            pltpu.VMEM((KCH, SW), jnp.float32),
            pltpu.VMEM((KCH, SW), jnp.float32),
            pltpu.VMEM((WCH, SW), jnp.float32),
            pltpu.VMEM((WCH, SW), jnp.float32),
            pltpu.SemaphoreType.DMA,
            pltpu.SemaphoreType.DMA,
            pltpu.SemaphoreType.DMA,
            pltpu.SemaphoreType.DMA,
            pltpu.SemaphoreType.DMA,
            pltpu.SemaphoreType.DMA,
            pltpu.SemaphoreType.DMA,
            pltpu.SemaphoreType.DMA,
            pltpu.VMEM_SHARED((NP, SW), jnp.float32),
        ],
    )
    def k(y_hbm, src_hbm, dst_hbm, out_hbm, src_v, dst_v, rows0_v, rows1_v,
          rows2_v, rows3_v, zb_v, wb_v, gsem0, gsem1, gsem2, gsem3,
          ssem0, ssem1, ssem2, ssem3, acc_sh):
        c = lax.axis_index("c")
        s = lax.axis_index("s")

        pltpu.sync_copy(src_hbm.at[s], src_v)
        pltpu.sync_copy(dst_hbm.at[s], dst_v)

        @pl.loop(0, WCH)
        def _(r):
            for kk in range(SW // 16):
                zb_v[r, pl.ds(16 * kk, 16)] = jnp.zeros((16,), jnp.float32)

        for jj in range(nspc):
            slab = c * nspc + jj

            def g_start(j, buf, sem):
                pltpu.async_copy(y_hbm.at[slab].at[src_v.at[j]], buf, sem)

            def g_wait(buf, sem):
                pltpu.make_async_copy(
                    y_hbm.at[slab].at[src_v.at[0]], buf, sem).wait()

            def s_start(j, buf, sem):
                pltpu.async_copy(buf, acc_sh.at[dst_v.at[j]], sem, add=True)

            def s_wait(buf, sem):
                pltpu.make_async_copy(
                    buf, acc_sh.at[dst_v.at[0]], sem).wait()

            for p in range(RSTR // WCH):
                pltpu.sync_copy(zb_v, acc_sh.at[pl.ds(s * RSTR + p * WCH, WCH)])
            plsc.subcore_barrier()

            bufs = (rows0_v, rows1_v, rows2_v, rows3_v)
            gsems = (gsem0, gsem1, gsem2, gsem3)
            ssems = (ssem0, ssem1, ssem2, ssem3)

            g_start(0, bufs[0], gsems[0])
            g_start(1, bufs[1], gsems[1])
            g_start(2, bufs[2], gsems[2])
            for cc in range(4):
                bk = cc % 4
                g_wait(bufs[bk], gsems[bk])
                s_start(cc, bufs[bk], ssems[bk])
                nxt = (cc + 3) % 4
                if cc == 0:
                    g_start(3, bufs[3], gsems[3])
                else:
                    s_wait(bufs[nxt], ssems[nxt])
                    g_start(cc + 3, bufs[nxt], gsems[nxt])

            @pl.loop(1, NCH // 4 - 1)
            def _(h):
                for kk in range(4):
                    cc = 4 * h + kk
                    g_wait(bufs[kk], gsems[kk])
                    s_start(cc, bufs[kk], ssems[kk])
                    nxt = (kk + 3) % 4
                    s_wait(bufs[nxt], ssems[nxt])
                    g_start(cc + 3, bufs[nxt], gsems[nxt])

            for kk in range(4):
                cc = NCH - 4 + kk
                g_wait(bufs[kk], gsems[kk])
                s_start(cc, bufs[kk], ssems[kk])
                if kk == 0:
                    nxt = 3
                    s_wait(bufs[nxt], ssems[nxt])
                    g_start(cc + 3, bufs[nxt], gsems[nxt])
            for kk in range(4):
                s_wait(bufs[kk], ssems[kk])
            plsc.subcore_barrier()
            for p in range(RSTR // WCH):
                r0 = s * RSTR + p * WCH
                pltpu.sync_copy(acc_sh.at[pl.ds(r0, WCH)], wb_v)
                pltpu.sync_copy(wb_v, out_hbm.at[slab].at[pl.ds(r0, WCH)])
            plsc.subcore_barrier()

    return k(y_slabs, src_r, dst_r)


def _edge_mask_call(node_mask_flat, src_flat, dst_flat):
    ep_w = E // (NT * NC)
    nchunk = ep_w // 16 + 1

    @functools.partial(
        pl.kernel,
        out_type=jax.ShapeDtypeStruct((E,), jnp.float32),
        mesh=_MESH,
        compiler_params=_sc_params(),
        scratch_types=[
            pltpu.VMEM((N,), jnp.float32),
            pltpu.VMEM((nchunk * 16,), jnp.int32),
            pltpu.VMEM((nchunk * 16,), jnp.int32),
            pltpu.VMEM((nchunk * 16,), jnp.float32),
        ],
    )
    def k(nm_hbm, src_hbm, dst_hbm, out_hbm, tab_v, src_v, dst_v, out_v):
        c = lax.axis_index("c")
        s = lax.axis_index("s")
        wid = s * NC + c
        base = wid * ep_w

        pltpu.sync_copy(nm_hbm, tab_v)
        src_v[pl.ds(ep_w - 8, 16)] = jnp.zeros((16,), jnp.int32)
        dst_v[pl.ds(ep_w - 8, 16)] = jnp.zeros((16,), jnp.int32)
        pltpu.sync_copy(src_hbm.at[pl.ds(base, ep_w)], src_v.at[pl.ds(0, ep_w)])
        pltpu.sync_copy(dst_hbm.at[pl.ds(base, ep_w)], dst_v.at[pl.ds(0, ep_w)])

        @pl.loop(0, nchunk)
        def _(i):
            sl = pl.ds(i * 16, 16)
            a = plsc.load_gather(tab_v, [src_v[sl]])
            b = plsc.load_gather(tab_v, [dst_v[sl]])
            out_v[sl] = 0.5 * (a + b)

        pltpu.sync_copy(out_v.at[pl.ds(0, ep_w)], out_hbm.at[pl.ds(base, ep_w)])

    return k(node_mask_flat, src_flat, dst_flat)


def _mm1_body(x_ref, w_ref, deg_ref, y1_ref, dis_ref):
    xw = jnp.dot(x_ref[...].astype(jnp.bfloat16), w_ref[...].astype(jnp.bfloat16),
                 preferred_element_type=jnp.float32)
    dis = lax.rsqrt(deg_ref[:, 0:1])
    y = xw * dis
    for kk in range(8):
        y1_ref[kk] = y[:, SW * kk:SW * (kk + 1)]
    dis_ref[...] = dis


def _mm1_call(x, W1, deg16):
    return pl.pallas_call(
        _mm1_body,
        grid=(GRID,),
        in_specs=[
            pl.BlockSpec((RB, 1280), lambda i: (i, 0)),
            pl.BlockSpec((1280, 512), lambda i: (0, 0)),
            pl.BlockSpec((RB, 16), lambda i: (i, 0)),
        ],
        out_specs=[
            pl.BlockSpec((8, RB, SW), lambda i: (0, i, 0)),
            pl.BlockSpec((RB, 1), lambda i: (i, 0)),
        ],
        out_shape=[
            jax.ShapeDtypeStruct((8, N, SW), jnp.float32),
            jax.ShapeDtypeStruct((N, 1), jnp.float32),
        ],
    )(x, W1, deg16)


def _mm2_body(agg_ref, dis_ref, w2_ref, b1_ref, y2_ref):
    agg = jnp.concatenate([agg_ref[kk] for kk in range(8)], axis=-1)
    dis = dis_ref[...]
    h1 = jnp.maximum(agg * dis + b1_ref[...], 0.0)
    xw2 = jnp.dot(h1.astype(jnp.bfloat16), w2_ref[...].astype(jnp.bfloat16),
                  preferred_element_type=jnp.float32)
    y2 = xw2 * dis
    for kk in range(4):
        y2_ref[kk] = y2[:, SW * kk:SW * (kk + 1)]


def _mm2_call(agg1, dis, W2, b1r):
    return pl.pallas_call(
        _mm2_body,
        grid=(GRID,),
        in_specs=[
            pl.BlockSpec((8, RB, SW), lambda i: (0, i, 0)),
            pl.BlockSpec((RB, 1), lambda i: (i, 0)),
            pl.BlockSpec((512, 256), lambda i: (0, 0)),
            pl.BlockSpec((1, 512), lambda i: (0, 0)),
        ],
        out_specs=[pl.BlockSpec((4, RB, SW), lambda i: (0, i, 0))],
        out_shape=[jax.ShapeDtypeStruct((4, N, SW), jnp.float32)],
    )(agg1, dis, W2, b1r)[0]


def _head_body(agg_ref, dis_ref, b2_ref, wl_ref, bl_ref, eps_ref,
               nm_ref, kld_ref):
    agg = jnp.concatenate([agg_ref[kk] for kk in range(4)], axis=-1)
    dis = dis_ref[...]
    h2 = jnp.maximum(agg * dis + b2_ref[...], 0.0)
    pre = jnp.dot(h2, wl_ref[...], preferred_element_type=jnp.float32)
    pre = pre + bl_ref[...]
    pre = jnp.clip(pre, -10.0, 10.0)
    eps = eps_ref[...]
    gate = jnp.log(eps) - jnp.log(1.0 - eps) + pre
    nm = jax.nn.sigmoid(gate)
    nm_ref[...] = nm
    ee = 1e-08
    t = nm * jnp.log(nm / 0.5 + ee) + (1.0 - nm) * jnp.log((1.0 - nm) / 0.5 + ee)
    partial = jnp.full((1, 1), jnp.sum(t) * (1.0 / N), jnp.float32)

    @pl.when(pl.program_id(0) == 0)
    def _():
        kld_ref[...] = jnp.zeros((1, 1), jnp.float32)

    kld_ref[...] += partial


def _head_call(agg2, dis, b2r, Wl, blr, eps):
    return pl.pallas_call(
        _head_body,
        grid=(GRID,),
        in_specs=[
            pl.BlockSpec((4, RB, SW), lambda i: (0, i, 0)),
            pl.BlockSpec((RB, 1), lambda i: (i, 0)),
            pl.BlockSpec((1, 256), lambda i: (0, 0)),
            pl.BlockSpec((256, 1), lambda i: (0, 0)),
            pl.BlockSpec((1, 1), lambda i: (0, 0)),
            pl.BlockSpec((RB, 1), lambda i: (i, 0)),
        ],
        out_specs=[
            pl.BlockSpec((RB, 1), lambda i: (i, 0)),
            pl.BlockSpec((1, 1), lambda i: (0, 0)),
        ],
        out_shape=[
            jax.ShapeDtypeStruct((N, 1), jnp.float32),
            jax.ShapeDtypeStruct((1, 1), jnp.float32),
        ],
    )(agg2, dis, b2r, Wl, blr, eps)


@jax.jit
def kernel(x, edge_index, W1, b1, W2, b2, Wl, bl, eps):
    src = edge_index[0]
    dst = edge_index[1]
    loops = jnp.arange(N, dtype=jnp.int32)
    pada = jnp.arange(E2 - E - N, dtype=jnp.int32)
    padi = pada % N
    padd = N + (pada % (NP - N))
    src_r = jnp.concatenate([src, loops, padi]).reshape(NT, NCH, KCH)
    dst_r = jnp.concatenate([dst, loops, padd]).reshape(NT, NCH, KCH)

    deg16 = _deg_call(dst_r).reshape(NP, 16)
    y1, dis = _mm1_call(x, W1, deg16)
    agg1 = _agg_call(y1, src_r, dst_r, 8)
    y2 = _mm2_call(agg1, dis, W2, b1.reshape(1, 512))
    agg2 = _agg_call(y2, src_r, dst_r, 4)
    node_mask, kld = _head_call(agg2, dis, b2.reshape(1, 256), Wl,
                                bl.reshape(1, 1), eps)
    edge_mask = _edge_mask_call(node_mask.reshape(N), src, dst)
    return kld[0, 0], node_mask, edge_mask[:, None]


# speedup vs baseline: 2.4490x; 1.0017x over previous
import dataclasses
import functools

import jax
import jax.numpy as jnp
from jax import lax
from jax.experimental import pallas as pl
from jax.experimental.pallas import tpu as pltpu
from jax.experimental.pallas import tpu_sc as plsc

N = 10000
NP = 10240
E = 160000
E2 = 176000
NT = 16
NC = 2
EP_T = E2 // NT
KCH = 125
NCH = EP_T // KCH
RSTR = NP // NT
WCH = 128
SW = 64
RB = 400
GRID = N // RB


def _sc_params():
    cp = pltpu.CompilerParams(use_tc_tiling_on_sc=False)
    if "needs_layout_passes" in pltpu.CompilerParams.__dataclass_fields__:
        cp = dataclasses.replace(cp, needs_layout_passes=False)
    return cp


_MESH = plsc.VectorSubcoreMesh(core_axis_name="c", subcore_axis_name="s")


def _deg_call(dst_r):
    @functools.partial(
        pl.kernel,
        out_type=jax.ShapeDtypeStruct((NT, RSTR, 16), jnp.float32),
        mesh=_MESH,
        compiler_params=_sc_params(),
        scratch_types=[
            pltpu.VMEM((NCH, KCH), jnp.int32),
            pltpu.VMEM((KCH, 16), jnp.float32),
            pltpu.VMEM((RSTR, 16), jnp.float32),
            pltpu.VMEM_SHARED((NP, 16), jnp.float32),
        ],
    )
    def k(dst_hbm, deg_hbm, idx_v, ones_v, stripe_v, acc_sh):
        c = lax.axis_index("c")
        s = lax.axis_index("s")

        @pl.loop(0, RSTR)
        def _(r):
            stripe_v[r] = jnp.zeros((16,), jnp.float32)

        @pl.loop(0, KCH)
        def _(r):
            ones_v[r] = jnp.ones((16,), jnp.float32)

        pltpu.sync_copy(stripe_v, acc_sh.at[pl.ds(s * RSTR, RSTR)])
        pltpu.sync_copy(dst_hbm.at[s], idx_v)
        plsc.subcore_barrier()

        @pl.loop(0, NCH)
        def _(j):
            pltpu.sync_copy(ones_v, acc_sh.at[idx_v.at[j]], add=True)

        plsc.subcore_barrier()

        @pl.when(c == 0)
        def _():
            pltpu.sync_copy(acc_sh.at[pl.ds(s * RSTR, RSTR)], stripe_v)
            pltpu.sync_copy(stripe_v, deg_hbm.at[s])

    return k(dst_r)


def _agg_call(y_slabs, src_r, dst_r, n_slabs):
    nspc = n_slabs // NC

    @functools.partial(
        pl.kernel,
        out_type=jax.ShapeDtypeStruct((n_slabs, NP, SW), jnp.float32),
        mesh=_MESH,
        compiler_params=_sc_params(),
        scratch_types=[
            pltpu.VMEM((NCH, KCH), jnp.int32),
            pltpu.VMEM((NCH, KCH), jnp.int32),
            pltpu.VMEM((KCH, SW), jnp.float32),
            pltpu.VMEM((KCH, SW), jnp.float32),
            pltpu.VMEM((KCH, SW), jnp.float32),
            pltpu.VMEM((KCH, SW), jnp.float32),
            pltpu.VMEM((WCH, SW), jnp.float32),
            pltpu.VMEM((WCH, SW), jnp.float32),
            pltpu.SemaphoreType.DMA,
            pltpu.SemaphoreType.DMA,
            pltpu.SemaphoreType.DMA,
            pltpu.SemaphoreType.DMA,
            pltpu.SemaphoreType.DMA,
            pltpu.SemaphoreType.DMA,
            pltpu.SemaphoreType.DMA,
            pltpu.SemaphoreType.DMA,
            pltpu.VMEM_SHARED((NP, SW), jnp.float32),
        ],
    )
    def k(y_hbm, src_hbm, dst_hbm, out_hbm, src_v, dst_v, rows0_v, rows1_v,
          rows2_v, rows3_v, zb_v, wb_v,
          gsem0, gsem1, gsem2, gsem3,
          ssem0, ssem1, ssem2, ssem3, acc_sh):
        c = lax.axis_index("c")
        s = lax.axis_index("s")

        pltpu.sync_copy(src_hbm.at[s], src_v)
        pltpu.sync_copy(dst_hbm.at[s], dst_v)

        @pl.loop(0, WCH)
        def _(r):
            for kk in range(SW // 16):
                zb_v[r, pl.ds(16 * kk, 16)] = jnp.zeros((16,), jnp.float32)

        for jj in range(nspc):
            slab = c * nspc + jj

            def g_start(j, buf, sem):
                pltpu.async_copy(y_hbm.at[slab].at[src_v.at[j]], buf, sem)

            def g_wait(buf, sem):
                pltpu.make_async_copy(
                    y_hbm.at[slab].at[src_v.at[0]], buf, sem).wait()

            def s_start(j, buf, sem):
                pltpu.async_copy(buf, acc_sh.at[dst_v.at[j]], sem, add=True)

            def s_wait(buf, sem):
                pltpu.make_async_copy(
                    buf, acc_sh.at[dst_v.at[0]], sem).wait()

            for p in range(RSTR // WCH):
                pltpu.sync_copy(zb_v, acc_sh.at[pl.ds(s * RSTR + p * WCH, WCH)])
            plsc.subcore_barrier()

            D = 4
            bufs = (rows0_v, rows1_v, rows2_v, rows3_v)
            gsems = (gsem0, gsem1, gsem2, gsem3)
            ssems = (ssem0, ssem1, ssem2, ssem3)

            for cc in range(D - 1):
                g_start(cc, bufs[cc], gsems[cc])
            for cc in range(D):
                bk = cc % D
                g_wait(bufs[bk], gsems[bk])
                s_start(cc, bufs[bk], ssems[bk])
                nxt = (cc + D - 1) % D
                if cc == 0:
                    g_start(D - 1, bufs[D - 1], gsems[D - 1])
                else:
                    s_wait(bufs[nxt], ssems[nxt])
                    g_start(cc + D - 1, bufs[nxt], gsems[nxt])

            @pl.loop(1, NCH // D - 1)
            def _(h):
                for kk in range(D):
                    cc = D * h + kk
                    g_wait(bufs[kk], gsems[kk])
                    s_start(cc, bufs[kk], ssems[kk])
                    nxt = (kk + D - 1) % D
                    s_wait(bufs[nxt], ssems[nxt])
                    g_start(cc + D - 1, bufs[nxt], gsems[nxt])

            for kk in range(D):
                cc = NCH - D + kk
                g_wait(bufs[kk], gsems[kk])
                s_start(cc, bufs[kk], ssems[kk])
                if kk == 0:
                    nxt = D - 1
                    s_wait(bufs[nxt], ssems[nxt])
                    g_start(cc + D - 1, bufs[nxt], gsems[nxt])
            for kk in range(D):
                s_wait(bufs[kk], ssems[kk])
            plsc.subcore_barrier()
            for p in range(RSTR // WCH):
                r0 = s * RSTR + p * WCH
                pltpu.sync_copy(acc_sh.at[pl.ds(r0, WCH)], wb_v)
                pltpu.sync_copy(wb_v, out_hbm.at[slab].at[pl.ds(r0, WCH)])
            plsc.subcore_barrier()

    return k(y_slabs, src_r, dst_r)


def _edge_mask_call(node_mask_flat, src_flat, dst_flat):
    ep_w = E // (NT * NC)
    nchunk = ep_w // 16 + 1

    @functools.partial(
        pl.kernel,
        out_type=jax.ShapeDtypeStruct((E,), jnp.float32),
        mesh=_MESH,
        compiler_params=_sc_params(),
        scratch_types=[
            pltpu.VMEM((N,), jnp.float32),
            pltpu.VMEM((nchunk * 16,), jnp.int32),
            pltpu.VMEM((nchunk * 16,), jnp.int32),
            pltpu.VMEM((nchunk * 16,), jnp.float32),
        ],
    )
    def k(nm_hbm, src_hbm, dst_hbm, out_hbm, tab_v, src_v, dst_v, out_v):
        c = lax.axis_index("c")
        s = lax.axis_index("s")
        wid = s * NC + c
        base = wid * ep_w

        pltpu.sync_copy(nm_hbm, tab_v)
        src_v[pl.ds(ep_w - 8, 16)] = jnp.zeros((16,), jnp.int32)
        dst_v[pl.ds(ep_w - 8, 16)] = jnp.zeros((16,), jnp.int32)
        pltpu.sync_copy(src_hbm.at[pl.ds(base, ep_w)], src_v.at[pl.ds(0, ep_w)])
        pltpu.sync_copy(dst_hbm.at[pl.ds(base, ep_w)], dst_v.at[pl.ds(0, ep_w)])

        @pl.loop(0, nchunk)
        def _(i):
            sl = pl.ds(i * 16, 16)
            a = plsc.load_gather(tab_v, [src_v[sl]])
            b = plsc.load_gather(tab_v, [dst_v[sl]])
            out_v[sl] = 0.5 * (a + b)

        pltpu.sync_copy(out_v.at[pl.ds(0, ep_w)], out_hbm.at[pl.ds(base, ep_w)])

    return k(node_mask_flat, src_flat, dst_flat)


def _mm1_body(x_ref, w_ref, deg_ref, y1_ref, dis_ref):
    xw = jnp.dot(x_ref[...].astype(jnp.bfloat16), w_ref[...].astype(jnp.bfloat16),
                 preferred_element_type=jnp.float32)
    dis = lax.rsqrt(deg_ref[:, 0:1])
    y = xw * dis
    for kk in range(8):
        y1_ref[kk] = y[:, SW * kk:SW * (kk + 1)]
    dis_ref[...] = dis


def _mm1_call(x, W1, deg16):
    return pl.pallas_call(
        _mm1_body,
        grid=(GRID,),
        in_specs=[
            pl.BlockSpec((RB, 1280), lambda i: (i, 0)),
            pl.BlockSpec((1280, 512), lambda i: (0, 0)),
            pl.BlockSpec((RB, 16), lambda i: (i, 0)),
        ],
        out_specs=[
            pl.BlockSpec((8, RB, SW), lambda i: (0, i, 0)),
            pl.BlockSpec((RB, 1), lambda i: (i, 0)),
        ],
        out_shape=[
            jax.ShapeDtypeStruct((8, N, SW), jnp.float32),
            jax.ShapeDtypeStruct((N, 1), jnp.float32),
        ],
    )(x, W1, deg16)


def _mm2_body(agg_ref, dis_ref, w2_ref, b1_ref, y2_ref):
    agg = jnp.concatenate([agg_ref[kk] for kk in range(8)], axis=-1)
    dis = dis_ref[...]
    h1 = jnp.maximum(agg * dis + b1_ref[...], 0.0)
    xw2 = jnp.dot(h1.astype(jnp.bfloat16), w2_ref[...].astype(jnp.bfloat16),
                  preferred_element_type=jnp.float32)
    y2 = xw2 * dis
    for kk in range(4):
        y2_ref[kk] = y2[:, SW * kk:SW * (kk + 1)]


def _mm2_call(agg1, dis, W2, b1r):
    return pl.pallas_call(
        _mm2_body,
        grid=(GRID,),
        in_specs=[
            pl.BlockSpec((8, RB, SW), lambda i: (0, i, 0)),
            pl.BlockSpec((RB, 1), lambda i: (i, 0)),
            pl.BlockSpec((512, 256), lambda i: (0, 0)),
            pl.BlockSpec((1, 512), lambda i: (0, 0)),
        ],
        out_specs=[pl.BlockSpec((4, RB, SW), lambda i: (0, i, 0))],
        out_shape=[jax.ShapeDtypeStruct((4, N, SW), jnp.float32)],
    )(agg1, dis, W2, b1r)[0]


def _head_body(agg_ref, dis_ref, b2_ref, wl_ref, bl_ref, eps_ref,
               nm_ref, kld_ref):
    agg = jnp.concatenate([agg_ref[kk] for kk in range(4)], axis=-1)
    dis = dis_ref[...]
    h2 = jnp.maximum(agg * dis + b2_ref[...], 0.0)
    pre = jnp.dot(h2, wl_ref[...], preferred_element_type=jnp.float32)
    pre = pre + bl_ref[...]
    pre = jnp.clip(pre, -10.0, 10.0)
    eps = eps_ref[...]
    gate = jnp.log(eps) - jnp.log(1.0 - eps) + pre
    nm = jax.nn.sigmoid(gate)
    nm_ref[...] = nm
    ee = 1e-08
    t = nm * jnp.log(nm / 0.5 + ee) + (1.0 - nm) * jnp.log((1.0 - nm) / 0.5 + ee)
    partial = jnp.full((1, 1), jnp.sum(t) * (1.0 / N), jnp.float32)

    @pl.when(pl.program_id(0) == 0)
    def _():
        kld_ref[...] = jnp.zeros((1, 1), jnp.float32)

    kld_ref[...] += partial


def _head_call(agg2, dis, b2r, Wl, blr, eps):
    return pl.pallas_call(
        _head_body,
        grid=(GRID,),
        in_specs=[
            pl.BlockSpec((4, RB, SW), lambda i: (0, i, 0)),
            pl.BlockSpec((RB, 1), lambda i: (i, 0)),
            pl.BlockSpec((1, 256), lambda i: (0, 0)),
            pl.BlockSpec((256, 1), lambda i: (0, 0)),
            pl.BlockSpec((1, 1), lambda i: (0, 0)),
            pl.BlockSpec((RB, 1), lambda i: (i, 0)),
        ],
        out_specs=[
            pl.BlockSpec((RB, 1), lambda i: (i, 0)),
            pl.BlockSpec((1, 1), lambda i: (0, 0)),
        ],
        out_shape=[
            jax.ShapeDtypeStruct((N, 1), jnp.float32),
            jax.ShapeDtypeStruct((1, 1), jnp.float32),
        ],
    )(agg2, dis, b2r, Wl, blr, eps)


@jax.jit
def kernel(x, edge_index, W1, b1, W2, b2, Wl, bl, eps):
    src = edge_index[0]
    dst = edge_index[1]
    loops = jnp.arange(N, dtype=jnp.int32)
    pada = jnp.arange(E2 - E - N, dtype=jnp.int32)
    padi = pada % N
    padd = N + (pada % (NP - N))
    src_r = jnp.concatenate([src, loops, padi]).reshape(NT, NCH, KCH)
    dst_r = jnp.concatenate([dst, loops, padd]).reshape(NT, NCH, KCH)

    deg16 = _deg_call(dst_r).reshape(NP, 16)
    y1, dis = _mm1_call(x, W1, deg16)
    agg1 = _agg_call(y1, src_r, dst_r, 8)
    y2 = _mm2_call(agg1, dis, W2, b1.reshape(1, 512))
    agg2 = _agg_call(y2, src_r, dst_r, 4)
    node_mask, kld = _head_call(agg2, dis, b2.reshape(1, 256), Wl,
                                bl.reshape(1, 1), eps)
    edge_mask = _edge_mask_call(node_mask.reshape(N), src, dst)
    return kld[0, 0], node_mask, edge_mask[:, None]
